# Initial kernel scaffold; baseline (speedup 1.0000x reference)
import jax
import jax.numpy as jnp
from jax.experimental import pallas as pl


def kernel(x, edge_index, W1, a1_src, a1_dst, b1, W2, a2_src, a2_dst, b2):
    raise NotImplementedError("write your pallas kernel here")


# speedup vs baseline: 48.8468x; 48.8468x over previous
import functools

import jax
import jax.numpy as jnp
from jax import lax
from jax.experimental import pallas as pl
from jax.experimental.pallas import tpu as pltpu
from jax.experimental.pallas import tpu_sc as plsc

N = 10000
E = 320000
D = 128

NC = 2
NS = 16
L = 16
NW = NC * NS

CHUNK = 128
NCHW = 80
EP = NW * NCHW * CHUNK
TOTCH = EP // CHUNK
NPAD = 10112
RPS = NPAD // NS
ZR = RPS // 4

_mesh = plsc.VectorSubcoreMesh(core_axis_name="c", subcore_axis_name="s")


def _take(x, idx):
    return jnp.take_along_axis(x, idx, axis=0, mode="promise_in_bounds")


def _lrelu(t):
    return jnp.where(t > 0, t, 0.2 * t)


@functools.partial(
    pl.kernel,
    out_type=jax.ShapeDtypeStruct((NC, NPAD, 80), jnp.float32),
    mesh=_mesh,
    compiler_params=pltpu.CompilerParams(use_tc_tiling_on_sc=False),
    scratch_types=[
        pltpu.VMEM((NCHW, CHUNK), jnp.int32),
        pltpu.VMEM((NCHW, CHUNK), jnp.int32),
        pltpu.VMEM((CHUNK, 80), jnp.float32),
        pltpu.VMEM((CHUNK, 16), jnp.float32),
        pltpu.VMEM((16,), jnp.float32),
        pltpu.VMEM((ZR, 80), jnp.float32),
        pltpu.MemorySpace.VMEM_SHARED((NPAD, 80), jnp.float32),
        pltpu.SemaphoreType.DMA,
        pltpu.SemaphoreType.DMA,
    ],
)
def _sc_layer1(htab, satab, sidx, didx, cvec, part,
               idx_s, idx_d, rows, dsa, cbuf, zbuf, accum, sem1, sem2):
    cid = lax.axis_index("c")
    sid = lax.axis_index("s")
    wid = cid * NS + sid

    zero16 = jnp.zeros((L,), jnp.float32)
    iota = jax.lax.iota(jnp.int32, L)
    idx_lo = iota & 7
    idx_hi = idx_lo + 8

    def _zrow(r, _):
        def _zcol(k, _):
            zbuf[r, pl.ds(k * L, L)] = zero16
            return 0
        return lax.fori_loop(0, 80 // L, _zcol, 0)
    lax.fori_loop(0, ZR, _zrow, 0)
    base = sid * RPS
    for z in range(4):
        pltpu.sync_copy(zbuf, accum.at[pl.ds(base + z * ZR, ZR), :])

    pltpu.sync_copy(cvec, cbuf)
    pltpu.sync_copy(sidx.at[pl.ds(wid * NCHW, NCHW), :], idx_s)
    pltpu.sync_copy(didx.at[pl.ds(wid * NCHW, NCHW), :], idx_d)
    creg = cbuf[...]

    plsc.subcore_barrier()

    def _chunk(j, _):
        g1 = pltpu.async_copy(htab.at[idx_s.at[j]], rows, sem1)
        g2 = pltpu.async_copy(satab.at[idx_d.at[j]], dsa, sem2)
        g1.wait()
        g2.wait()

        def _edge(e, _):
            st = rows[e, pl.ds(64, L)]
            dt = dsa[e, :]
            t = _take(st, idx_lo) + _take(dt, idx_hi)
            w2 = jnp.exp(_lrelu(t) - creg)
            rows[e, pl.ds(64, L)] = w2
            for k in range(4):
                wk = _take(w2, (iota >> 3) + 2 * k)
                rows[e, pl.ds(k * L, L)] = rows[e, pl.ds(k * L, L)] * wk
            return 0

        lax.fori_loop(0, CHUNK, _edge, 0)
        pltpu.sync_copy(rows, accum.at[idx_d.at[j]], add=True)
        return 0

    lax.fori_loop(0, NCHW, _chunk, 0)

    plsc.subcore_barrier()
    pltpu.sync_copy(accum.at[pl.ds(base, RPS), :],
                    part.at[cid, pl.ds(base, RPS), :])


@functools.partial(
    pl.kernel,
    out_type=jax.ShapeDtypeStruct((NC, NPAD, 16), jnp.float32),
    mesh=_mesh,
    compiler_params=pltpu.CompilerParams(use_tc_tiling_on_sc=False),
    scratch_types=[
        pltpu.VMEM((NCHW, CHUNK), jnp.int32),
        pltpu.VMEM((NCHW, CHUNK), jnp.int32),
        pltpu.VMEM((CHUNK, 16), jnp.float32),
        pltpu.VMEM((CHUNK, 16), jnp.float32),
        pltpu.VMEM((16,), jnp.float32),
        pltpu.VMEM((ZR, 16), jnp.float32),
        pltpu.MemorySpace.VMEM_SHARED((NPAD, 16), jnp.float32),
        pltpu.SemaphoreType.DMA,
        pltpu.SemaphoreType.DMA,
    ],
)
def _sc_layer2(tab2, sidx, didx, cvec, part,
               idx_s, idx_d, rows, dsa, cbuf, zbuf, accum, sem1, sem2):
    cid = lax.axis_index("c")
    sid = lax.axis_index("s")
    wid = cid * NS + sid

    zero16 = jnp.zeros((L,), jnp.float32)
    one16 = jnp.ones((L,), jnp.float32)
    iota = jax.lax.iota(jnp.int32, L)
    idx7 = jnp.full((L,), 7, jnp.int32)
    idx8 = jnp.full((L,), 8, jnp.int32)

    def _zrow(r, _):
        zbuf[r, :] = zero16
        return 0
    lax.fori_loop(0, ZR, _zrow, 0)
    base = sid * RPS
    for z in range(4):
        pltpu.sync_copy(zbuf, accum.at[pl.ds(base + z * ZR, ZR), :])

    pltpu.sync_copy(cvec, cbuf)
    pltpu.sync_copy(sidx.at[pl.ds(wid * NCHW, NCHW), :], idx_s)
    pltpu.sync_copy(didx.at[pl.ds(wid * NCHW, NCHW), :], idx_d)
    creg = cbuf[...]

    plsc.subcore_barrier()

    def _chunk(j, _):
        g1 = pltpu.async_copy(tab2.at[idx_s.at[j]], rows, sem1)
        g2 = pltpu.async_copy(tab2.at[idx_d.at[j]], dsa, sem2)
        g1.wait()
        g2.wait()

        def _edge(e, _):
            s = rows[e, :]
            d = dsa[e, :]
            t = _take(s, idx7) + _take(d, idx8)
            w = jnp.exp(_lrelu(t) - creg)
            sel = jnp.where(iota == 7, one16,
                            jnp.where(iota >= 8, zero16, s))
            rows[e, :] = w * sel
            return 0

        lax.fori_loop(0, CHUNK, _edge, 0)
        pltpu.sync_copy(rows, accum.at[idx_d.at[j]], add=True)
        return 0

    lax.fori_loop(0, NCHW, _chunk, 0)

    plsc.subcore_barrier()
    pltpu.sync_copy(accum.at[pl.ds(base, RPS), :],
                    part.at[cid, pl.ds(base, RPS), :])


_R1 = 1000
_G = N // _R1


def _k1_body(x_ref, w1_ref, a1s_ref, a1d_ref, htab_ref, satab_ref, mx_ref):
    h = jnp.dot(x_ref[...], w1_ref[...])
    as_ = jnp.dot(h, a1s_ref[...])
    ad_ = jnp.dot(h, a1d_ref[...])
    htab_ref[:, 0:64] = h
    htab_ref[:, 64:72] = as_
    htab_ref[:, 72:80] = ad_
    satab_ref[:, 0:8] = as_
    satab_ref[:, 8:16] = ad_
    m_as = jnp.max(as_, axis=0)
    m_ad = jnp.max(ad_, axis=0)
    mx_ref[...] = jnp.concatenate([m_as, m_ad]).reshape(1, 1, 16)


def _k1(x, W1, A1s, A1d):
    return pl.pallas_call(
        _k1_body,
        grid=(_G,),
        in_specs=[
            pl.BlockSpec((_R1, D), lambda i: (i, 0)),
            pl.BlockSpec((D, 64), lambda i: (0, 0)),
            pl.BlockSpec((64, 8), lambda i: (0, 0)),
            pl.BlockSpec((64, 8), lambda i: (0, 0)),
        ],
        out_specs=[
            pl.BlockSpec((_R1, 80), lambda i: (i, 0)),
            pl.BlockSpec((_R1, 16), lambda i: (i, 0)),
            pl.BlockSpec((1, 1, 16), lambda i: (i, 0, 0)),
        ],
        out_shape=[
            jax.ShapeDtypeStruct((N, 80), jnp.float32),
            jax.ShapeDtypeStruct((N, 16), jnp.float32),
            jax.ShapeDtypeStruct((_G, 1, 16), jnp.float32),
        ],
    )(x, W1, A1s, A1d)


def _k2_body(p0_ref, p1_ref, htab_ref, c1_ref, w2_ref, a2s_ref, a2d_ref,
             b1_ref, r8_ref, tab2_ref, mx_ref):
    ht = htab_ref[...]
    h = ht[:, 0:64]
    as_ = ht[:, 64:72]
    ad_ = ht[:, 72:80]
    c1 = c1_ref[...][:, 0:8]
    ws = jnp.exp(_lrelu(as_ + ad_) - c1)
    r8 = r8_ref[...]
    num = p0_ref[:, 0:64] + p1_ref[:, 0:64] + h * jnp.dot(ws, r8)
    den = p0_ref[:, 64:72] + p1_ref[:, 64:72] + ws + 1e-16
    h1 = num / jnp.dot(den, r8) + b1_ref[...]
    h2 = jnp.dot(h1, w2_ref[...])
    as2 = jnp.dot(h2, a2s_ref[...])
    ad2 = jnp.dot(h2, a2d_ref[...])
    tab2_ref[:, 0:7] = h2
    tab2_ref[:, 7:8] = as2
    tab2_ref[:, 8:9] = ad2
    tab2_ref[:, 9:16] = jnp.zeros_like(tab2_ref[:, 9:16])
    lane = lax.broadcasted_iota(jnp.int32, (1, 1, 16), 2)
    m = jnp.where(lane == 0, jnp.max(as2),
                  jnp.where(lane == 1, jnp.max(ad2), -1e30))
    mx_ref[...] = m


def _k2(p0, p1, htab, c1row, W2, a2sT, a2dT, b1row, R8):
    return pl.pallas_call(
        _k2_body,
        grid=(_G,),
        in_specs=[
            pl.BlockSpec((_R1, 80), lambda i: (i, 0)),
            pl.BlockSpec((_R1, 80), lambda i: (i, 0)),
            pl.BlockSpec((_R1, 80), lambda i: (i, 0)),
            pl.BlockSpec((1, 16), lambda i: (0, 0)),
            pl.BlockSpec((64, 7), lambda i: (0, 0)),
            pl.BlockSpec((7, 1), lambda i: (0, 0)),
            pl.BlockSpec((7, 1), lambda i: (0, 0)),
            pl.BlockSpec((1, 64), lambda i: (0, 0)),
            pl.BlockSpec((8, 64), lambda i: (0, 0)),
        ],
        out_specs=[
            pl.BlockSpec((_R1, 16), lambda i: (i, 0)),
            pl.BlockSpec((1, 1, 16), lambda i: (i, 0, 0)),
        ],
        out_shape=[
            jax.ShapeDtypeStruct((N, 16), jnp.float32),
            jax.ShapeDtypeStruct((_G, 1, 16), jnp.float32),
        ],
    )(p0, p1, htab, c1row, W2, a2sT, a2dT, b1row, R8)


def _k3_body(p0_ref, p1_ref, tab2_ref, c2_ref, b2_ref, out_ref):
    t = tab2_ref[...]
    h2 = t[:, 0:7]
    as2 = t[:, 7:8]
    ad2 = t[:, 8:9]
    c2 = c2_ref[...][0, 0]
    ws = jnp.exp(_lrelu(as2 + ad2) - c2)
    num = p0_ref[:, 0:7] + p1_ref[:, 0:7] + h2 * ws
    den = p0_ref[:, 7:8] + p1_ref[:, 7:8] + ws + 1e-16
    z = num / den + b2_ref[...]
    zm = z - jnp.max(z, axis=1, keepdims=True)
    out_ref[...] = zm - jnp.log(jnp.sum(jnp.exp(zm), axis=1, keepdims=True))


def _k3(p0, p1, tab2, c2row, b2row):
    return pl.pallas_call(
        _k3_body,
        grid=(_G,),
        in_specs=[
            pl.BlockSpec((_R1, 16), lambda i: (i, 0)),
            pl.BlockSpec((_R1, 16), lambda i: (i, 0)),
            pl.BlockSpec((_R1, 16), lambda i: (i, 0)),
            pl.BlockSpec((1, 16), lambda i: (0, 0)),
            pl.BlockSpec((1, 7), lambda i: (0, 0)),
        ],
        out_specs=pl.BlockSpec((_R1, 7), lambda i: (i, 0)),
        out_shape=jax.ShapeDtypeStruct((N, 7), jnp.float32),
    )(p0, p1, tab2, c2row, b2row)


def kernel(x, edge_index, W1, a1_src, a1_dst, b1, W2, a2_src, a2_dst, b2):
    f32 = jnp.float32
    rows64 = jnp.arange(64)
    A1s = jnp.zeros((64, 8), f32).at[rows64, rows64 // 8].set(a1_src.reshape(64))
    A1d = jnp.zeros((64, 8), f32).at[rows64, rows64 // 8].set(a1_dst.reshape(64))
    R8 = jnp.repeat(jnp.eye(8, dtype=f32), 8, axis=1)

    src = edge_index[0]
    dst = edge_index[1]
    idt = edge_index.dtype
    srcp = jnp.concatenate([src, jnp.zeros((EP - E,), idt)]).reshape(TOTCH, CHUNK)
    dstp = jnp.concatenate([dst, jnp.full((EP - E,), N, idt)]).reshape(TOTCH, CHUNK)
    srcp = srcp.astype(jnp.int32)
    dstp = dstp.astype(jnp.int32)

    htab, satab, mx1 = _k1(x, W1, A1s, A1d)
    m1 = jnp.max(mx1.reshape(_G, 16), axis=0)
    c1 = _lrelu(m1[0:8] + m1[8:16])
    cvec1 = jnp.concatenate([c1, c1])
    part1 = _sc_layer1(htab, satab, srcp, dstp, cvec1)
    tab2, mx2 = _k2(part1[0], part1[1], htab, cvec1.reshape(1, 16),
                    W2, a2_src.reshape(7, 1), a2_dst.reshape(7, 1),
                    b1.reshape(1, 64), R8)

    m2 = jnp.max(mx2.reshape(_G, 16), axis=0)
    c2 = _lrelu(m2[0] + m2[1])
    cvec2 = jnp.full((16,), c2, f32)
    part2 = _sc_layer2(tab2, srcp, dstp, cvec2)
    return _k3(part2[0], part2[1], tab2, cvec2.reshape(1, 16),
               b2.reshape(1, 7))
